# Initial kernel scaffold; baseline (speedup 1.0000x reference)
import jax
import jax.numpy as jnp
from jax.experimental import pallas as pl


def kernel(x, edge_index, batch, dist, vec_hat, W_rbf, b_rbf, We_in, be_in, We_h, be_h, We_out, be_out, Wf_in, bf_in, Wf_h, bf_h, Wf_out, bf_out):
    raise NotImplementedError("write your pallas kernel here")


# speedup vs baseline: 3.0130x; 3.0130x over previous
import functools

import jax
import jax.numpy as jnp
from jax import lax
from jax.experimental import pallas as pl
from jax.experimental.pallas import tpu as pltpu
from jax.experimental.pallas import tpu_sc as plsc

D = 128
NG = 50
RADIUS = 12.0
N_NODES = 10000
N_EDGES = 320000
N_GRAPHS = 64

NC, NS = 2, 16
NW = NC * NS
EPT = 10240
E_PAD = NW * EPT
GCH = 128
NGCH = EPT // GCH

EB = 30080
ACC_LEN = 30720
SLC = ACC_LEN // NS
SCH = 1024
NSCH = EPT // SCH
NROW = 4 * SCH // 128

_mesh = functools.partial(
    plsc.VectorSubcoreMesh, core_axis_name="c", subcore_axis_name="s",
    num_cores=NC, num_subcores=NS)


def _gather_body(x_hbm, srcg_hbm, dstg_hbm, xs_hbm, xd_hbm,
                 idxs_v, idxd_v, rb0, rb1, sg0, sg1, ss0, ss1):
    wid = lax.axis_index("s") * NC + lax.axis_index("c")
    base_r = wid * (NGCH)
    pltpu.sync_copy(srcg_hbm.at[pl.ds(base_r, NGCH)], idxs_v)
    pltpu.sync_copy(dstg_hbm.at[pl.ds(base_r, NGCH)], idxd_v)
    base_e = wid * EPT

    rbufs = (rb0, rb1)
    gsems = (sg0, sg1)
    ssems = (ss0, ss1)
    nch = 2 * NGCH

    def chunk_refs(j):
        if j < NGCH:
            return idxs_v.at[j], xs_hbm.at[pl.ds(base_e + j * GCH, GCH)]
        jj = j - NGCH
        return idxd_v.at[jj], xd_hbm.at[pl.ds(base_e + jj * GCH, GCH)]

    idx0, _ = chunk_refs(0)
    g = [None, None]
    s = [None, None]
    g[0] = pltpu.async_copy(x_hbm.at[idx0], rbufs[0], gsems[0])
    for j in range(nch):
        b = j % 2
        g[b].wait()
        _, out_ref = chunk_refs(j)
        s[b] = pltpu.async_copy(rbufs[b], out_ref, ssems[b])
        if j + 1 < nch:
            if j >= 1:
                s[1 - b].wait()
            idx_ref, _ = chunk_refs(j + 1)
            g[1 - b] = pltpu.async_copy(x_hbm.at[idx_ref], rbufs[1 - b],
                                        gsems[1 - b])
    s[(nch - 2) % 2].wait()
    s[(nch - 1) % 2].wait()


def _sc_gather(x, srcg2d, dstg2d):
    body = pl.kernel(
        _gather_body,
        out_type=(jax.ShapeDtypeStruct((E_PAD, D), jnp.float32),
                  jax.ShapeDtypeStruct((E_PAD, D), jnp.float32)),
        mesh=_mesh(),
        scratch_types=[
            pltpu.VMEM((NGCH, GCH), jnp.int32),
            pltpu.VMEM((NGCH, GCH), jnp.int32),
            pltpu.VMEM((GCH, D), jnp.float32),
            pltpu.VMEM((GCH, D), jnp.float32),
            pltpu.SemaphoreType.DMA,
            pltpu.SemaphoreType.DMA,
            pltpu.SemaphoreType.DMA,
            pltpu.SemaphoreType.DMA,
        ],
    )
    return body(x, srcg2d, dstg2d)


_SPACING = RADIUS / (NG - 1)
_COEFF = -0.5 / (_SPACING * _SPACING)


def _mlp_body(xs_ref, xd_ref, d_ref, w_ref, aux_ref, se_ref, sf_ref):
    xs = xs_ref[...]
    xd = xd_ref[...]
    d = d_ref[...]
    offs = aux_ref[0:1, :]
    g = jnp.exp(_COEFF * (d - offs) ** 2)

    def dot(a, b):
        return jax.lax.dot_general(a, b, (((1,), (0,)), ((), ())),
                                   preferred_element_type=jnp.float32)

    he = dot(xs, w_ref[0]) + dot(xd, w_ref[1]) + dot(g, w_ref[2]) \
        + aux_ref[1:2, :]
    he = he * jax.nn.sigmoid(he)
    h2 = he + (lambda t: t * jax.nn.sigmoid(t))(dot(he, w_ref[3])
                                                + aux_ref[2:3, :])
    se = jnp.sum(h2 * aux_ref[3:4, :], axis=1, keepdims=True) + aux_ref[7, 0]

    hf = dot(xs, w_ref[4]) + dot(xd, w_ref[5]) + dot(g, w_ref[6]) \
        + aux_ref[4:5, :]
    hf = hf * jax.nn.sigmoid(hf)
    f2 = hf + (lambda t: t * jax.nn.sigmoid(t))(dot(hf, w_ref[7])
                                                + aux_ref[5:6, :])
    sf = jnp.sum(f2 * aux_ref[6:7, :], axis=1, keepdims=True) + aux_ref[7, 1]

    se_ref[...] = se
    sf_ref[...] = sf


def _tc_mlp(xs, xd, dist2d, wstack, aux, be=2048):
    n = xs.shape[0]
    grid = (n // be,)
    return pl.pallas_call(
        _mlp_body,
        grid=grid,
        in_specs=[
            pl.BlockSpec((be, D), lambda i: (i, 0)),
            pl.BlockSpec((be, D), lambda i: (i, 0)),
            pl.BlockSpec((be, 1), lambda i: (i, 0)),
            pl.BlockSpec((8, D, D), lambda i: (0, 0, 0)),
            pl.BlockSpec((8, D), lambda i: (0, 0)),
        ],
        out_specs=[
            pl.BlockSpec((be, 1), lambda i: (i, 0)),
            pl.BlockSpec((be, 1), lambda i: (i, 0)),
        ],
        out_shape=[
            jax.ShapeDtypeStruct((n, 1), jnp.float32),
            jax.ShapeDtypeStruct((n, 1), jnp.float32),
        ],
    )(xs, xd, dist2d, wstack, aux)


def _scatter_body(se_hbm, sf_hbm, vhx_hbm, vhy_hbm, vhz_hbm, srcs2d_hbm,
                  batch_hbm, out_hbm,
                  se_v, sf_v, vx_v, vy_v, vz_v, src2d_v, b2d_v,
                  vals_v, idx_v, zbuf, acc, sem):
    cid = lax.axis_index("c")
    sid = lax.axis_index("s")
    wid = sid * NC + cid

    zeros16 = jnp.zeros((16,), jnp.float32)
    for i in range(SLC // 16):
        zbuf[pl.ds(i * 16, 16)] = zeros16
    pltpu.sync_copy(zbuf, acc.at[pl.ds(sid * SLC, SLC)])
    plsc.subcore_barrier()

    base = wid * EPT
    wrow = wid * (EPT // 128)
    rps = SCH // 128

    def chunk(k, carry):
        off = base + k * SCH
        pltpu.sync_copy(se_hbm.at[pl.ds(off, SCH)], se_v)
        pltpu.sync_copy(sf_hbm.at[pl.ds(off, SCH)], sf_v)
        pltpu.sync_copy(vhx_hbm.at[pl.ds(off, SCH)], vx_v)
        pltpu.sync_copy(vhy_hbm.at[pl.ds(off, SCH)], vy_v)
        pltpu.sync_copy(vhz_hbm.at[pl.ds(off, SCH)], vz_v)
        pltpu.sync_copy(srcs2d_hbm.at[pl.ds(wrow + k * rps, rps)], src2d_v)

        bc = [pltpu.async_copy(batch_hbm.at[src2d_v.at[j]], b2d_v.at[j], sem)
              for j in range(rps)]
        for c in bc:
            c.wait()

        for r in range(NROW):
            for half in range(2):
                grp = 2 * r + half
                eo = grp * 16
                irow, icol = grp // 8, (grp % 8) * 16
                co = half * 64
                s16 = src2d_v[irow, pl.ds(icol, 16)]
                b16 = b2d_v[irow, pl.ds(icol, 16)]
                se16 = se_v[pl.ds(eo, 16)]
                sf16 = sf_v[pl.ds(eo, 16)]
                i3 = s16 * 3
                vals_v[r, pl.ds(co + 0, 16)] = sf16 * vx_v[pl.ds(eo, 16)]
                vals_v[r, pl.ds(co + 16, 16)] = sf16 * vy_v[pl.ds(eo, 16)]
                vals_v[r, pl.ds(co + 32, 16)] = sf16 * vz_v[pl.ds(eo, 16)]
                vals_v[r, pl.ds(co + 48, 16)] = se16
                idx_v[r, pl.ds(co + 0, 16)] = i3
                idx_v[r, pl.ds(co + 16, 16)] = i3 + 1
                idx_v[r, pl.ds(co + 32, 16)] = i3 + 2
                idx_v[r, pl.ds(co + 48, 16)] = b16 + EB

        copies = [pltpu.async_copy(vals_v.at[r], acc.at[idx_v.at[r]], sem,
                                   add=True)
                  for r in range(NROW)]
        for c in copies:
            c.wait()
        return carry

    lax.fori_loop(0, NSCH, chunk, 0)

    plsc.subcore_barrier()
    pltpu.sync_copy(acc.at[pl.ds(sid * SLC, SLC)],
                    out_hbm.at[pl.ds(cid * ACC_LEN + sid * SLC, SLC)])


def _sc_scatter(se_flat, sf_flat, vhx, vhy, vhz, srcs, batch_pad):
    body = pl.kernel(
        _scatter_body,
        out_type=jax.ShapeDtypeStruct((NC * ACC_LEN,), jnp.float32),
        mesh=_mesh(),
        scratch_types=[
            pltpu.VMEM((SCH,), jnp.float32),
            pltpu.VMEM((SCH,), jnp.float32),
            pltpu.VMEM((SCH,), jnp.float32),
            pltpu.VMEM((SCH,), jnp.float32),
            pltpu.VMEM((SCH,), jnp.float32),
            pltpu.VMEM((SCH // 128, 128), jnp.int32),
            pltpu.VMEM((SCH // 128, 128), jnp.int32),
            pltpu.VMEM((NROW, 128), jnp.float32),
            pltpu.VMEM((NROW, 128), jnp.int32),
            pltpu.VMEM((SLC,), jnp.float32),
            pltpu.VMEM_SHARED((ACC_LEN,), jnp.float32),
            pltpu.SemaphoreType.DMA,
        ],
    )
    return body(se_flat, sf_flat, vhx, vhy, vhz, srcs, batch_pad)


def kernel(x, edge_index, batch, dist, vec_hat,
           W_rbf, b_rbf,
           We_in, be_in, We_h, be_h, We_out, be_out,
           Wf_in, bf_in, Wf_h, bf_h, Wf_out, bf_out):
    pad = E_PAD - N_EDGES
    src = edge_index[0]
    dst = edge_index[1]
    srcg2d = jnp.pad(src, (0, pad)).reshape(E_PAD // 128, 128)
    dstg2d = jnp.pad(dst, (0, pad)).reshape(E_PAD // 128, 128)
    srcs2d = jnp.pad(src, (0, pad),
                     constant_values=N_NODES).reshape(E_PAD // 128, 128)
    dist2d = jnp.pad(dist, (0, pad)).reshape(E_PAD, 1)
    vhp = jnp.pad(vec_hat, ((0, pad), (0, 0)))
    batch_pad = jnp.pad(batch, (0, 16), constant_values=N_GRAPHS)

    offs = jnp.pad(jnp.linspace(0.0, RADIUS, NG), (0, D - NG))
    def fold(W_in, b_in):
        W1, W2, W3 = W_in[:D], W_in[D:2 * D], W_in[2 * D:]
        Wg = jnp.pad(W_rbf @ W3, ((0, D - NG), (0, 0)))
        bias = b_in + b_rbf @ W3
        return W1, W2, Wg, bias

    We1, We2, WgE, biasE = fold(We_in, be_in)
    Wf1, Wf2, WgF, biasF = fold(Wf_in, bf_in)
    weO = We_out[:, 0] / (60.0 ** 2)
    wfO = Wf_out[:, 0] / 60.0
    wstack = jnp.stack([We1, We2, WgE, We_h, Wf1, Wf2, WgF, Wf_h])
    scal = jnp.zeros((D,), jnp.float32)
    scal = scal.at[0].set(be_out[0] / (60.0 ** 2)).at[1].set(bf_out[0] / 60.0)
    aux = jnp.stack([offs, biasE, be_h, weO, biasF, bf_h, wfO, scal])

    xs, xd = _sc_gather(x, srcg2d, dstg2d)
    se, sf = _tc_mlp(xs, xd, dist2d, wstack, aux)
    partials = _sc_scatter(se.reshape(E_PAD), sf.reshape(E_PAD),
                           vhp[:, 0], vhp[:, 1], vhp[:, 2], srcs2d, batch_pad)
    p = partials[:ACC_LEN] + partials[ACC_LEN:]
    energy = p[EB:EB + N_GRAPHS].reshape(N_GRAPHS, 1)
    forces = p[:3 * N_NODES].reshape(N_NODES, 3)
    return (energy, forces)


# speedup vs baseline: 3.1788x; 1.0550x over previous
import functools

import jax
import jax.numpy as jnp
from jax import lax
from jax.experimental import pallas as pl
from jax.experimental.pallas import tpu as pltpu
from jax.experimental.pallas import tpu_sc as plsc

D = 128
NG = 50
RADIUS = 12.0
N_NODES = 10000
N_EDGES = 320000
N_GRAPHS = 64

NC, NS = 2, 16
NW = NC * NS
EPT = 10240
E_PAD = NW * EPT
GCH = 128
NGCH = EPT // GCH

EB = 30080
ACC_LEN = 30720
SLC = ACC_LEN // NS
SCH = 1024
NSCH = EPT // SCH
NROW = 4 * SCH // 128

_mesh = functools.partial(
    plsc.VectorSubcoreMesh, core_axis_name="c", subcore_axis_name="s",
    num_cores=NC, num_subcores=NS)


_NBUF = 4


def _gather_body(x_hbm, srcg_hbm, dstg_hbm, xs_hbm, xd_hbm,
                 idxs_v, idxd_v, *rest):
    rbufs = rest[:_NBUF]
    gsems = rest[_NBUF:2 * _NBUF]
    ssems = rest[2 * _NBUF:3 * _NBUF]
    wid = lax.axis_index("s") * NC + lax.axis_index("c")
    base_r = wid * (NGCH)
    pltpu.sync_copy(srcg_hbm.at[pl.ds(base_r, NGCH)], idxs_v)
    pltpu.sync_copy(dstg_hbm.at[pl.ds(base_r, NGCH)], idxd_v)
    base_e = wid * EPT
    nch = 2 * NGCH

    def chunk_refs(j):
        if j < NGCH:
            return idxs_v.at[j], xs_hbm.at[pl.ds(base_e + j * GCH, GCH)]
        jj = j - NGCH
        return idxd_v.at[jj], xd_hbm.at[pl.ds(base_e + jj * GCH, GCH)]

    g = [None] * _NBUF
    s = [None] * _NBUF
    for j in range(nch + _NBUF):
        b = j % _NBUF
        if j < nch:
            if s[b] is not None:
                s[b].wait()
            idx_ref, _ = chunk_refs(j)
            g[b] = pltpu.async_copy(x_hbm.at[idx_ref], rbufs[b], gsems[b])
        jd = j - _NBUF + 1
        if 0 <= jd < nch:
            bd = jd % _NBUF
            g[bd].wait()
            _, out_ref = chunk_refs(jd)
            s[bd] = pltpu.async_copy(rbufs[bd], out_ref, ssems[bd])
    for t in range(_NBUF):
        if s[t] is not None:
            s[t].wait()


def _sc_gather(x, srcg2d, dstg2d):
    body = pl.kernel(
        _gather_body,
        out_type=(jax.ShapeDtypeStruct((E_PAD, D), jnp.float32),
                  jax.ShapeDtypeStruct((E_PAD, D), jnp.float32)),
        mesh=_mesh(),
        scratch_types=[
            pltpu.VMEM((NGCH, GCH), jnp.int32),
            pltpu.VMEM((NGCH, GCH), jnp.int32),
        ] + [pltpu.VMEM((GCH, D), jnp.float32) for _ in range(_NBUF)]
          + [pltpu.SemaphoreType.DMA for _ in range(2 * _NBUF)],
    )
    return body(x, srcg2d, dstg2d)


_SPACING = RADIUS / (NG - 1)
_COEFF = -0.5 / (_SPACING * _SPACING)


def _mlp_body(xs_ref, xd_ref, d_ref, w_ref, aux_ref, se_ref, sf_ref):
    xs = xs_ref[...]
    xd = xd_ref[...]
    d = d_ref[...]
    offs = aux_ref[0:1, :]
    g = jnp.exp(_COEFF * (d - offs) ** 2)

    def dot(a, b):
        return jax.lax.dot_general(a, b, (((1,), (0,)), ((), ())),
                                   preferred_element_type=jnp.float32)

    he = dot(xs, w_ref[0]) + dot(xd, w_ref[1]) + dot(g, w_ref[2]) \
        + aux_ref[1:2, :]
    he = he * jax.nn.sigmoid(he)
    h2 = he + (lambda t: t * jax.nn.sigmoid(t))(dot(he, w_ref[3])
                                                + aux_ref[2:3, :])
    se = jnp.sum(h2 * aux_ref[3:4, :], axis=1, keepdims=True) + aux_ref[7, 0]

    hf = dot(xs, w_ref[4]) + dot(xd, w_ref[5]) + dot(g, w_ref[6]) \
        + aux_ref[4:5, :]
    hf = hf * jax.nn.sigmoid(hf)
    f2 = hf + (lambda t: t * jax.nn.sigmoid(t))(dot(hf, w_ref[7])
                                                + aux_ref[5:6, :])
    sf = jnp.sum(f2 * aux_ref[6:7, :], axis=1, keepdims=True) + aux_ref[7, 1]

    se_ref[...] = se
    sf_ref[...] = sf


def _tc_mlp(xs, xd, dist2d, wstack, aux, be=2048):
    n = xs.shape[0]
    grid = (n // be,)
    return pl.pallas_call(
        _mlp_body,
        grid=grid,
        in_specs=[
            pl.BlockSpec((be, D), lambda i: (i, 0)),
            pl.BlockSpec((be, D), lambda i: (i, 0)),
            pl.BlockSpec((be, 1), lambda i: (i, 0)),
            pl.BlockSpec((8, D, D), lambda i: (0, 0, 0)),
            pl.BlockSpec((8, D), lambda i: (0, 0)),
        ],
        out_specs=[
            pl.BlockSpec((be, 1), lambda i: (i, 0)),
            pl.BlockSpec((be, 1), lambda i: (i, 0)),
        ],
        out_shape=[
            jax.ShapeDtypeStruct((n, 1), jnp.float32),
            jax.ShapeDtypeStruct((n, 1), jnp.float32),
        ],
    )(xs, xd, dist2d, wstack, aux)


def _scatter_body(se_hbm, sf_hbm, vhx_hbm, vhy_hbm, vhz_hbm, srcs2d_hbm,
                  batch_hbm, out_hbm,
                  se_v, sf_v, vx_v, vy_v, vz_v, src2d_v, b2d_v,
                  vals_v, idx_v, zbuf, acc, sem):
    cid = lax.axis_index("c")
    sid = lax.axis_index("s")
    wid = sid * NC + cid

    zeros16 = jnp.zeros((16,), jnp.float32)
    for i in range(SLC // 16):
        zbuf[pl.ds(i * 16, 16)] = zeros16
    pltpu.sync_copy(zbuf, acc.at[pl.ds(sid * SLC, SLC)])
    plsc.subcore_barrier()

    base = wid * EPT
    wrow = wid * (EPT // 128)
    rps = SCH // 128

    def chunk(k, carry):
        off = base + k * SCH
        pltpu.sync_copy(se_hbm.at[pl.ds(off, SCH)], se_v)
        pltpu.sync_copy(sf_hbm.at[pl.ds(off, SCH)], sf_v)
        pltpu.sync_copy(vhx_hbm.at[pl.ds(off, SCH)], vx_v)
        pltpu.sync_copy(vhy_hbm.at[pl.ds(off, SCH)], vy_v)
        pltpu.sync_copy(vhz_hbm.at[pl.ds(off, SCH)], vz_v)
        pltpu.sync_copy(srcs2d_hbm.at[pl.ds(wrow + k * rps, rps)], src2d_v)

        bc = [pltpu.async_copy(batch_hbm.at[src2d_v.at[j]], b2d_v.at[j], sem)
              for j in range(rps)]
        for c in bc:
            c.wait()

        for r in range(NROW):
            for half in range(2):
                grp = 2 * r + half
                eo = grp * 16
                irow, icol = grp // 8, (grp % 8) * 16
                co = half * 64
                s16 = src2d_v[irow, pl.ds(icol, 16)]
                b16 = b2d_v[irow, pl.ds(icol, 16)]
                se16 = se_v[pl.ds(eo, 16)]
                sf16 = sf_v[pl.ds(eo, 16)]
                i3 = s16 * 3
                vals_v[r, pl.ds(co + 0, 16)] = sf16 * vx_v[pl.ds(eo, 16)]
                vals_v[r, pl.ds(co + 16, 16)] = sf16 * vy_v[pl.ds(eo, 16)]
                vals_v[r, pl.ds(co + 32, 16)] = sf16 * vz_v[pl.ds(eo, 16)]
                vals_v[r, pl.ds(co + 48, 16)] = se16
                idx_v[r, pl.ds(co + 0, 16)] = i3
                idx_v[r, pl.ds(co + 16, 16)] = i3 + 1
                idx_v[r, pl.ds(co + 32, 16)] = i3 + 2
                idx_v[r, pl.ds(co + 48, 16)] = b16 + EB

        copies = [pltpu.async_copy(vals_v.at[r], acc.at[idx_v.at[r]], sem,
                                   add=True)
                  for r in range(NROW)]
        for c in copies:
            c.wait()
        return carry

    lax.fori_loop(0, NSCH, chunk, 0)

    plsc.subcore_barrier()
    pltpu.sync_copy(acc.at[pl.ds(sid * SLC, SLC)],
                    out_hbm.at[pl.ds(cid * ACC_LEN + sid * SLC, SLC)])


def _sc_scatter(se_flat, sf_flat, vhx, vhy, vhz, srcs, batch_pad):
    body = pl.kernel(
        _scatter_body,
        out_type=jax.ShapeDtypeStruct((NC * ACC_LEN,), jnp.float32),
        mesh=_mesh(),
        scratch_types=[
            pltpu.VMEM((SCH,), jnp.float32),
            pltpu.VMEM((SCH,), jnp.float32),
            pltpu.VMEM((SCH,), jnp.float32),
            pltpu.VMEM((SCH,), jnp.float32),
            pltpu.VMEM((SCH,), jnp.float32),
            pltpu.VMEM((SCH // 128, 128), jnp.int32),
            pltpu.VMEM((SCH // 128, 128), jnp.int32),
            pltpu.VMEM((NROW, 128), jnp.float32),
            pltpu.VMEM((NROW, 128), jnp.int32),
            pltpu.VMEM((SLC,), jnp.float32),
            pltpu.VMEM_SHARED((ACC_LEN,), jnp.float32),
            pltpu.SemaphoreType.DMA,
        ],
    )
    return body(se_flat, sf_flat, vhx, vhy, vhz, srcs, batch_pad)


def kernel(x, edge_index, batch, dist, vec_hat,
           W_rbf, b_rbf,
           We_in, be_in, We_h, be_h, We_out, be_out,
           Wf_in, bf_in, Wf_h, bf_h, Wf_out, bf_out):
    pad = E_PAD - N_EDGES
    src = edge_index[0]
    dst = edge_index[1]
    srcg2d = jnp.pad(src, (0, pad)).reshape(E_PAD // 128, 128)
    dstg2d = jnp.pad(dst, (0, pad)).reshape(E_PAD // 128, 128)
    srcs2d = jnp.pad(src, (0, pad),
                     constant_values=N_NODES).reshape(E_PAD // 128, 128)
    dist2d = jnp.pad(dist, (0, pad)).reshape(E_PAD, 1)
    vhp = jnp.pad(vec_hat, ((0, pad), (0, 0)))
    batch_pad = jnp.pad(batch, (0, 16), constant_values=N_GRAPHS)

    offs = jnp.pad(jnp.linspace(0.0, RADIUS, NG), (0, D - NG))
    def fold(W_in, b_in):
        W1, W2, W3 = W_in[:D], W_in[D:2 * D], W_in[2 * D:]
        Wg = jnp.pad(W_rbf @ W3, ((0, D - NG), (0, 0)))
        bias = b_in + b_rbf @ W3
        return W1, W2, Wg, bias

    We1, We2, WgE, biasE = fold(We_in, be_in)
    Wf1, Wf2, WgF, biasF = fold(Wf_in, bf_in)
    weO = We_out[:, 0] / (60.0 ** 2)
    wfO = Wf_out[:, 0] / 60.0
    wstack = jnp.stack([We1, We2, WgE, We_h, Wf1, Wf2, WgF, Wf_h])
    scal = jnp.zeros((D,), jnp.float32)
    scal = scal.at[0].set(be_out[0] / (60.0 ** 2)).at[1].set(bf_out[0] / 60.0)
    aux = jnp.stack([offs, biasE, be_h, weO, biasF, bf_h, wfO, scal])

    xs, xd = _sc_gather(x, srcg2d, dstg2d)
    se, sf = _tc_mlp(xs, xd, dist2d, wstack, aux)
    partials = _sc_scatter(se.reshape(E_PAD), sf.reshape(E_PAD),
                           vhp[:, 0], vhp[:, 1], vhp[:, 2], srcs2d, batch_pad)
    p = partials[:ACC_LEN] + partials[ACC_LEN:]
    energy = p[EB:EB + N_GRAPHS].reshape(N_GRAPHS, 1)
    forces = p[:3 * N_NODES].reshape(N_NODES, 3)
    return (energy, forces)


# speedup vs baseline: 6.7844x; 2.1342x over previous
import functools

import jax
import jax.numpy as jnp
from jax import lax
from jax.experimental import pallas as pl
from jax.experimental.pallas import tpu as pltpu
from jax.experimental.pallas import tpu_sc as plsc

D = 128
NG = 50
RADIUS = 12.0
N_NODES = 10000
N_EDGES = 320000
N_GRAPHS = 64

NC, NS = 2, 16
NW = NC * NS
EPT = 10240
E_PAD = NW * EPT
GCH = 128
NGCH = EPT // GCH

EB = 30080
ACC_LEN = 30720
SLC = ACC_LEN // NS
SCH = 1024
NSCH = EPT // SCH
NROW = 4 * SCH // 128

_mesh = functools.partial(
    plsc.VectorSubcoreMesh, core_axis_name="c", subcore_axis_name="s",
    num_cores=NC, num_subcores=NS)


_NBUF = 4


def _gather_body(x_hbm, srcg_hbm, dstg_hbm, xs_hbm, xd_hbm,
                 idxs_v, idxd_v, x_sp, *rest):
    rbufs = rest[:_NBUF]
    gsems = rest[_NBUF:2 * _NBUF]
    ssems = rest[2 * _NBUF:3 * _NBUF]
    sid = lax.axis_index("s")
    wid = sid * NC + lax.axis_index("c")
    base_r = wid * (NGCH)
    pltpu.sync_copy(srcg_hbm.at[pl.ds(base_r, NGCH)], idxs_v)
    pltpu.sync_copy(dstg_hbm.at[pl.ds(base_r, NGCH)], idxd_v)

    pltpu.sync_copy(x_hbm.at[pl.ds(sid * 632, 632)],
                    x_sp.at[pl.ds(sid * 632, 632)])
    plsc.subcore_barrier()

    base_e = wid * EPT
    nch = 2 * NGCH

    def chunk_refs(j):
        if j < NGCH:
            return idxs_v.at[j], xs_hbm.at[pl.ds(base_e + j * GCH, GCH)]
        jj = j - NGCH
        return idxd_v.at[jj], xd_hbm.at[pl.ds(base_e + jj * GCH, GCH)]

    g = [None] * _NBUF
    s = [None] * _NBUF
    for j in range(nch + _NBUF):
        b = j % _NBUF
        if j < nch:
            if s[b] is not None:
                s[b].wait()
            idx_ref, _ = chunk_refs(j)
            g[b] = pltpu.async_copy(x_sp.at[idx_ref], rbufs[b], gsems[b])
        jd = j - _NBUF + 1
        if 0 <= jd < nch:
            bd = jd % _NBUF
            g[bd].wait()
            _, out_ref = chunk_refs(jd)
            s[bd] = pltpu.async_copy(rbufs[bd], out_ref, ssems[bd])
    for t in range(_NBUF):
        if s[t] is not None:
            s[t].wait()


def _sc_gather(x, srcg2d, dstg2d):
    body = pl.kernel(
        _gather_body,
        out_type=(jax.ShapeDtypeStruct((E_PAD, D // 2), jnp.float32),
                  jax.ShapeDtypeStruct((E_PAD, D // 2), jnp.float32)),
        mesh=_mesh(),
        scratch_types=[
            pltpu.VMEM((NGCH, GCH), jnp.int32),
            pltpu.VMEM((NGCH, GCH), jnp.int32),
            pltpu.VMEM_SHARED((16 * 632, D // 2), jnp.float32),
        ] + [pltpu.VMEM((GCH, D // 2), jnp.float32) for _ in range(_NBUF)]
          + [pltpu.SemaphoreType.DMA for _ in range(2 * _NBUF)],
    )
    return body(x, srcg2d, dstg2d)


_SPACING = RADIUS / (NG - 1)
_COEFF = -0.5 / (_SPACING * _SPACING)


def _unpack(ref):
    u = lax.bitcast_convert_type(ref[...], jnp.int32)
    ev = lax.bitcast_convert_type(u << 16, jnp.float32)
    od = lax.bitcast_convert_type(u & jnp.int32(-65536), jnp.float32)
    return ev, od


def _mlp_body(xs_ref, xd_ref, d_ref, wb_ref, w_ref, aux_ref, se_ref, sf_ref):
    xse, xso = _unpack(xs_ref)
    xde, xdo = _unpack(xd_ref)
    xs = jnp.concatenate([xse, xso], axis=1).astype(jnp.bfloat16)
    xd = jnp.concatenate([xde, xdo], axis=1).astype(jnp.bfloat16)
    d = d_ref[...]
    offs = aux_ref[0:1, :]
    g = jnp.exp(_COEFF * (d - offs) ** 2)

    def dot(a, b):
        return jax.lax.dot_general(a, b, (((1,), (0,)), ((), ())),
                                   preferred_element_type=jnp.float32)

    he = dot(xs, wb_ref[0]) + dot(xd, wb_ref[1]) + dot(g, w_ref[0]) \
        + aux_ref[1:2, :]
    he = he * jax.nn.sigmoid(he)
    h2 = he + (lambda t: t * jax.nn.sigmoid(t))(dot(he, w_ref[1])
                                                + aux_ref[2:3, :])
    se = jnp.sum(h2 * aux_ref[3:4, :], axis=1, keepdims=True) + aux_ref[7, 0]

    hf = dot(xs, wb_ref[2]) + dot(xd, wb_ref[3]) + dot(g, w_ref[2]) \
        + aux_ref[4:5, :]
    hf = hf * jax.nn.sigmoid(hf)
    f2 = hf + (lambda t: t * jax.nn.sigmoid(t))(dot(hf, w_ref[3])
                                                + aux_ref[5:6, :])
    sf = jnp.sum(f2 * aux_ref[6:7, :], axis=1, keepdims=True) + aux_ref[7, 1]

    se_ref[...] = se
    sf_ref[...] = sf


def _tc_mlp(xs, xd, dist2d, wbstack, wstack, aux, be=2048):
    n = xs.shape[0]
    grid = (n // be,)
    return pl.pallas_call(
        _mlp_body,
        grid=grid,
        in_specs=[
            pl.BlockSpec((be, D // 2), lambda i: (i, 0)),
            pl.BlockSpec((be, D // 2), lambda i: (i, 0)),
            pl.BlockSpec((be, 1), lambda i: (i, 0)),
            pl.BlockSpec((4, D, D), lambda i: (0, 0, 0)),
            pl.BlockSpec((4, D, D), lambda i: (0, 0, 0)),
            pl.BlockSpec((8, D), lambda i: (0, 0)),
        ],
        out_specs=[
            pl.BlockSpec((be, 1), lambda i: (i, 0)),
            pl.BlockSpec((be, 1), lambda i: (i, 0)),
        ],
        out_shape=[
            jax.ShapeDtypeStruct((n, 1), jnp.float32),
            jax.ShapeDtypeStruct((n, 1), jnp.float32),
        ],
    )(xs, xd, dist2d, wbstack, wstack, aux)


def _scatter_body(se_hbm, sf_hbm, vhx_hbm, vhy_hbm, vhz_hbm, srcs2d_hbm,
                  batch_hbm, out_hbm,
                  se_v, sf_v, vx_v, vy_v, vz_v, src2d_v, b2d_v,
                  vals_v, idx_v, zbuf, acc, sem):
    cid = lax.axis_index("c")
    sid = lax.axis_index("s")
    wid = sid * NC + cid

    zeros16 = jnp.zeros((16,), jnp.float32)
    for i in range(SLC // 16):
        zbuf[pl.ds(i * 16, 16)] = zeros16
    pltpu.sync_copy(zbuf, acc.at[pl.ds(sid * SLC, SLC)])
    plsc.subcore_barrier()

    base = wid * EPT
    wrow = wid * (EPT // 128)
    rps = SCH // 128

    def chunk(k, carry):
        off = base + k * SCH
        pltpu.sync_copy(se_hbm.at[pl.ds(off, SCH)], se_v)
        pltpu.sync_copy(sf_hbm.at[pl.ds(off, SCH)], sf_v)
        pltpu.sync_copy(vhx_hbm.at[pl.ds(off, SCH)], vx_v)
        pltpu.sync_copy(vhy_hbm.at[pl.ds(off, SCH)], vy_v)
        pltpu.sync_copy(vhz_hbm.at[pl.ds(off, SCH)], vz_v)
        pltpu.sync_copy(srcs2d_hbm.at[pl.ds(wrow + k * rps, rps)], src2d_v)

        bc = [pltpu.async_copy(batch_hbm.at[src2d_v.at[j]], b2d_v.at[j], sem)
              for j in range(rps)]
        for c in bc:
            c.wait()

        for r in range(NROW):
            for half in range(2):
                grp = 2 * r + half
                eo = grp * 16
                irow, icol = grp // 8, (grp % 8) * 16
                co = half * 64
                s16 = src2d_v[irow, pl.ds(icol, 16)]
                b16 = b2d_v[irow, pl.ds(icol, 16)]
                se16 = se_v[pl.ds(eo, 16)]
                sf16 = sf_v[pl.ds(eo, 16)]
                i3 = s16 * 3
                vals_v[r, pl.ds(co + 0, 16)] = sf16 * vx_v[pl.ds(eo, 16)]
                vals_v[r, pl.ds(co + 16, 16)] = sf16 * vy_v[pl.ds(eo, 16)]
                vals_v[r, pl.ds(co + 32, 16)] = sf16 * vz_v[pl.ds(eo, 16)]
                vals_v[r, pl.ds(co + 48, 16)] = se16
                idx_v[r, pl.ds(co + 0, 16)] = i3
                idx_v[r, pl.ds(co + 16, 16)] = i3 + 1
                idx_v[r, pl.ds(co + 32, 16)] = i3 + 2
                idx_v[r, pl.ds(co + 48, 16)] = b16 + EB

        copies = [pltpu.async_copy(vals_v.at[r], acc.at[idx_v.at[r]], sem,
                                   add=True)
                  for r in range(NROW)]
        for c in copies:
            c.wait()
        return carry

    lax.fori_loop(0, NSCH, chunk, 0)

    plsc.subcore_barrier()
    pltpu.sync_copy(acc.at[pl.ds(sid * SLC, SLC)],
                    out_hbm.at[pl.ds(cid * ACC_LEN + sid * SLC, SLC)])


def _sc_scatter(se_flat, sf_flat, vhx, vhy, vhz, srcs, batch_pad):
    body = pl.kernel(
        _scatter_body,
        out_type=jax.ShapeDtypeStruct((NC * ACC_LEN,), jnp.float32),
        mesh=_mesh(),
        scratch_types=[
            pltpu.VMEM((SCH,), jnp.float32),
            pltpu.VMEM((SCH,), jnp.float32),
            pltpu.VMEM((SCH,), jnp.float32),
            pltpu.VMEM((SCH,), jnp.float32),
            pltpu.VMEM((SCH,), jnp.float32),
            pltpu.VMEM((SCH // 128, 128), jnp.int32),
            pltpu.VMEM((SCH // 128, 128), jnp.int32),
            pltpu.VMEM((NROW, 128), jnp.float32),
            pltpu.VMEM((NROW, 128), jnp.int32),
            pltpu.VMEM((SLC,), jnp.float32),
            pltpu.VMEM_SHARED((ACC_LEN,), jnp.float32),
            pltpu.SemaphoreType.DMA,
        ],
    )
    return body(se_flat, sf_flat, vhx, vhy, vhz, srcs, batch_pad)


def kernel(x, edge_index, batch, dist, vec_hat,
           W_rbf, b_rbf,
           We_in, be_in, We_h, be_h, We_out, be_out,
           Wf_in, bf_in, Wf_h, bf_h, Wf_out, bf_out):
    pad = E_PAD - N_EDGES
    src = edge_index[0]
    dst = edge_index[1]
    srcg2d = jnp.pad(src, (0, pad)).reshape(E_PAD // 128, 128)
    dstg2d = jnp.pad(dst, (0, pad)).reshape(E_PAD // 128, 128)
    srcs2d = jnp.pad(src, (0, pad),
                     constant_values=N_NODES).reshape(E_PAD // 128, 128)
    dist2d = jnp.pad(dist, (0, pad)).reshape(E_PAD, 1)
    vhp = jnp.pad(vec_hat, ((0, pad), (0, 0)))
    batch_pad = jnp.pad(batch, (0, 16), constant_values=N_GRAPHS)

    offs = jnp.pad(jnp.linspace(0.0, RADIUS, NG), (0, D - NG))
    def fold(W_in, b_in):
        W1, W2, W3 = W_in[:D], W_in[D:2 * D], W_in[2 * D:]
        Wg = jnp.pad(W_rbf @ W3, ((0, D - NG), (0, 0)))
        bias = b_in + b_rbf @ W3
        return W1, W2, Wg, bias

    We1, We2, WgE, biasE = fold(We_in, be_in)
    Wf1, Wf2, WgF, biasF = fold(Wf_in, bf_in)
    weO = We_out[:, 0] / (60.0 ** 2)
    wfO = Wf_out[:, 0] / 60.0
    def perm(w):
        return jnp.concatenate([w[0::2], w[1::2]], axis=0)
    wbstack = jnp.stack([perm(We1), perm(We2),
                         perm(Wf1), perm(Wf2)]).astype(jnp.bfloat16)
    wstack = jnp.stack([WgE, We_h, WgF, Wf_h])
    scal = jnp.zeros((D,), jnp.float32)
    scal = scal.at[0].set(be_out[0] / (60.0 ** 2)).at[1].set(bf_out[0] / 60.0)
    aux = jnp.stack([offs, biasE, be_h, weO, biasF, bf_h, wfO, scal])

    xpad = jnp.pad(x, ((0, 16 * 632 - N_NODES), (0, 0))).astype(jnp.bfloat16)
    xpk = lax.bitcast_convert_type(xpad.reshape(16 * 632, D // 2, 2),
                                   jnp.float32)
    xs, xd = _sc_gather(xpk, srcg2d, dstg2d)
    se, sf = _tc_mlp(xs, xd, dist2d, wbstack, wstack, aux)
    partials = _sc_scatter(se.reshape(E_PAD), sf.reshape(E_PAD),
                           vhp[:, 0], vhp[:, 1], vhp[:, 2], srcs2d, batch_pad)
    p = partials[:ACC_LEN] + partials[ACC_LEN:]
    energy = p[EB:EB + N_GRAPHS].reshape(N_GRAPHS, 1)
    forces = p[:3 * N_NODES].reshape(N_NODES, 3)
    return (energy, forces)
